# baseline (speedup 1.0000x reference)
import functools

import jax
import jax.numpy as jnp
from jax.experimental import pallas as pl
from jax.experimental.pallas import tpu as pltpu


def _edge_mlp_body(ea_ref, w1_ref, b1_ref, w2_ref, b2_ref, out_ref):
    h = jnp.maximum(
        jnp.dot(ea_ref[...], w1_ref[...], preferred_element_type=jnp.float32)
        + b1_ref[...][None, :],
        0.0,
    )
    out_ref[...] = (
        jnp.dot(h, w2_ref[...], preferred_element_type=jnp.float32)
        + b2_ref[...][None, :]
    )


def _edge_mlp(ea, p, block=4096):
    e, de = ea.shape
    h = p[0]["w"].shape[1]
    ho = p[1]["w"].shape[1]
    grid = (e // block,)
    return pl.pallas_call(
        _edge_mlp_body,
        grid=grid,
        in_specs=[
            pl.BlockSpec((block, de), lambda i: (i, 0)),
            pl.BlockSpec((de, h), lambda i: (0, 0)),
            pl.BlockSpec((h,), lambda i: (0,)),
            pl.BlockSpec((h, ho), lambda i: (0, 0)),
            pl.BlockSpec((ho,), lambda i: (0,)),
        ],
        out_specs=pl.BlockSpec((block, ho), lambda i: (i, 0)),
        out_shape=jax.ShapeDtypeStruct((e, ho), jnp.float32),
    )(ea, p[0]["w"], p[0]["b"], p[1]["w"], p[1]["b"])


def _apply_lin(p, x):
    return x @ p["w"] + p["b"]


def _mlp(ps, x):
    return _apply_lin(ps[1], jnp.maximum(_apply_lin(ps[0], x), 0.0))


def _gine(p, x, src, dst, ea, n):
    e = _apply_lin(p["lin"], ea)
    m = jnp.maximum(x[src] + e, 0.0)
    agg = jnp.zeros((n, x.shape[1]), dtype=x.dtype).at[dst].add(m)
    return _mlp(p["nn"], x + agg)


def kernel(x, edge_index, edge_attr, u, params):
    src = edge_index[0]
    dst = edge_index[1]
    n = x.shape[0]
    ea1 = _edge_mlp(edge_attr, params["em1"])
    h1 = jnp.concatenate(
        [_gine(params["c1"][i], x, src, dst, ea1, n) for i in range(3)], axis=1
    )
    x1 = jnp.maximum(_apply_lin(params["lin1"], h1), 0.0)
    ea2 = _edge_mlp(edge_attr, params["em2"])
    h2 = jnp.concatenate(
        [_gine(params["c2"][i], x1, src, dst, ea2, n) for i in range(3)], axis=1
    )
    x2 = jnp.maximum(_apply_lin(params["lin2"], h2), 0.0)
    pooled = jnp.mean(x2, axis=0, keepdims=True)
    return _apply_lin(params["fc"], jnp.concatenate([pooled, u], axis=1))


# speedup vs baseline: 1.9301x; 1.9301x over previous
import functools

import jax
import jax.numpy as jnp
from jax import lax
from jax.experimental import pallas as pl
from jax.experimental.pallas import tpu as pltpu
from jax.experimental.pallas import tpu_sc as plsc

_NPAD = 10240


def _edge_proj_body(*refs):
    nout = (len(refs) - 5) // 3
    attr_ref, w1_ref, b1_ref, w2_ref, b2_ref = refs[:5]
    wh_refs = refs[5:5 + nout]
    bh_refs = refs[5 + nout:5 + 2 * nout]
    out_refs = refs[5 + 2 * nout:]
    h = jnp.maximum(
        jnp.dot(attr_ref[...], w1_ref[...], preferred_element_type=jnp.float32)
        + b1_ref[...][None, :],
        0.0,
    )
    ea = (
        jnp.dot(h, w2_ref[...], preferred_element_type=jnp.float32)
        + b2_ref[...][None, :]
    )
    for wh_ref, bh_ref, out_ref in zip(wh_refs, bh_refs, out_refs):
        for c in range(2):
            out_ref[c, :, :] = (
                jnp.dot(ea, wh_ref[c], preferred_element_type=jnp.float32)
                + bh_ref[c][None, :]
            )


def _edge_proj(edge_attr, em, whs, bhs, block=3200):
    e, de = edge_attr.shape
    h = em[0]["w"].shape[1]
    w3 = whs[0].shape[2]
    nout = len(whs)
    return pl.pallas_call(
        _edge_proj_body,
        grid=(e // block,),
        in_specs=[
            pl.BlockSpec((block, de), lambda i: (i, 0)),
            pl.BlockSpec((de, h), lambda i: (0, 0)),
            pl.BlockSpec((h,), lambda i: (0,)),
            pl.BlockSpec((h, h), lambda i: (0, 0)),
            pl.BlockSpec((h,), lambda i: (0,)),
        ]
        + [pl.BlockSpec((2, h, w3), lambda i: (0, 0, 0))] * nout
        + [pl.BlockSpec((2, w3), lambda i: (0, 0))] * nout,
        out_specs=[pl.BlockSpec((2, block, w3), lambda i: (0, i, 0))] * nout,
        out_shape=[jax.ShapeDtypeStruct((2, e, w3), jnp.float32)] * nout,
    )(edge_attr, em[0]["w"], em[0]["b"], em[1]["w"], em[1]["b"], *whs, *bhs)


@functools.cache
def _make_sc_agg(n, e, w):
    w3 = 3 * w
    B = 80
    TILES = 16
    epw = e // TILES
    nchunks = epw // B
    npt = n // TILES
    ZR = 128
    assert epw * TILES == e and nchunks * B == epw
    assert npt * TILES == n and npt % ZR == 0 and npt % 8 == 0

    mesh = plsc.VectorSubcoreMesh(core_axis_name="c", subcore_axis_name="s")

    @functools.partial(
        pl.kernel,
        out_type=jax.ShapeDtypeStruct((2 * n, w3), jnp.float32),
        mesh=mesh,
        compiler_params=pltpu.CompilerParams(use_tc_tiling_on_sc=False),
        scratch_types=[
            pltpu.VMEM((B,), jnp.int32),
            pltpu.VMEM((B,), jnp.int32),
            pltpu.VMEM((B,), jnp.int32),
            pltpu.VMEM((B, w), jnp.float32),
            pltpu.VMEM((B, w3), jnp.float32),
            pltpu.VMEM((B, w3), jnp.float32),
            pltpu.VMEM((ZR, w3), jnp.float32),
            pltpu.VMEM_SHARED((n, w3), jnp.float32),
            pltpu.SemaphoreType.DMA,
        ],
    )
    def sc_agg(xh, eh, src, dst, out, src_v, dst_v, idx_v, xs_v, e_v, m_v,
               zbuf, agg_sh, sem):
        c = lax.axis_index("c")
        s = lax.axis_index("s")
        cn = c * n

        def zrow(r, carry):
            for g in range(w3 // 16):
                zbuf[r, pl.ds(g * 16, 16)] = jnp.zeros((16,), jnp.float32)
            return carry

        lax.fori_loop(0, ZR, zrow, 0)
        row0 = s * npt
        for j in range(npt // ZR):
            pltpu.sync_copy(zbuf, agg_sh.at[pl.ds(row0 + j * ZR, ZR)])
        plsc.subcore_barrier()

        def chunk(k, carry):
            base = s * epw + k * B
            pltpu.sync_copy(src.at[pl.ds(base, B)], src_v)
            pltpu.sync_copy(dst.at[pl.ds(base, B)], dst_v)
            for g in range(B // 16):
                idx_v[pl.ds(g * 16, 16)] = src_v[pl.ds(g * 16, 16)] + cn
            gcp = pltpu.async_copy(xh.at[idx_v], xs_v, sem)
            pltpu.sync_copy(eh.at[pl.ds(c * e + base, B)], e_v)
            gcp.wait()

            def edge_row(b, carry2):
                for g in range(w // 16):
                    xv = xs_v[b, pl.ds(g * 16, 16)]
                    for i in range(3):
                        col = i * w + g * 16
                        m_v[b, pl.ds(col, 16)] = jnp.maximum(
                            xv + e_v[b, pl.ds(col, 16)], 0.0
                        )
                return carry2

            lax.fori_loop(0, B, edge_row, 0)
            pltpu.sync_copy(m_v, agg_sh.at[dst_v], add=True)
            return carry

        lax.fori_loop(0, nchunks, chunk, 0)
        plsc.subcore_barrier()

        pltpu.sync_copy(agg_sh.at[pl.ds(row0, npt)],
                        out.at[pl.ds(cn + row0, npt)])

    return sc_agg


def _apply_lin(p, x):
    return x @ p["w"] + p["b"]


def _mlp(ps, x):
    return _apply_lin(ps[1], jnp.maximum(_apply_lin(ps[0], x), 0.0))


def _col_group_weights(convs, q, w):
    wh = jnp.concatenate(
        [convs[i]["lin"]["w"][:, q * w:(q + 1) * w] for i in range(3)], axis=1
    )
    bh = jnp.concatenate(
        [convs[i]["lin"]["b"][q * w:(q + 1) * w] for i in range(3)]
    )
    return wh, bh


def _gine_layer(x, src, dst, edge_attr, em, convs, lin_out):
    n, d = x.shape
    e = src.shape[0]
    w = 32
    ngrp = d // w
    ncall = ngrp // 2
    npad = _NPAD
    whs, bhs = [], []
    for r in range(ncall):
        pair = [_col_group_weights(convs, 2 * r + c, w) for c in range(2)]
        whs.append(jnp.stack([p[0] for p in pair]))
        bhs.append(jnp.stack([p[1] for p in pair]))
    ehs = _edge_proj(edge_attr, em, whs, bhs)
    if ncall == 1:
        ehs = [ehs] if not isinstance(ehs, (list, tuple)) else ehs
    xp = jnp.pad(x, ((0, npad - n), (0, 0)))
    xq = xp.reshape(npad, ngrp, w).transpose(1, 0, 2)
    sc = _make_sc_agg(npad, e, w)
    aggs = []
    for r in range(ncall):
        xh = xq[2 * r:2 * r + 2].reshape(2 * npad, w)
        aggs.append(sc(xh, ehs[r].reshape(2 * e, 3 * w), src, dst))
    outs = []
    for i in range(3):
        agg_i = jnp.concatenate(
            [
                aggs[q // 2][(q % 2) * npad:(q % 2) * npad + n,
                             i * w:(i + 1) * w]
                for q in range(ngrp)
            ],
            axis=1,
        )
        outs.append(_mlp(convs[i]["nn"], x + agg_i))
    h = jnp.concatenate(outs, axis=1)
    return jnp.maximum(_apply_lin(lin_out, h), 0.0)


def kernel(x, edge_index, edge_attr, u, params):
    src = edge_index[0]
    dst = edge_index[1]
    x1 = _gine_layer(x, src, dst, edge_attr, params["em1"], params["c1"],
                     params["lin1"])
    x2 = _gine_layer(x1, src, dst, edge_attr, params["em2"], params["c2"],
                     params["lin2"])
    pooled = jnp.mean(x2, axis=0, keepdims=True)
    return _apply_lin(params["fc"], jnp.concatenate([pooled, u], axis=1))


# speedup vs baseline: 2.9341x; 1.5202x over previous
import functools

import jax
import jax.numpy as jnp
from jax import lax
from jax.experimental import pallas as pl
from jax.experimental.pallas import tpu as pltpu
from jax.experimental.pallas import tpu_sc as plsc

_NPAD = 10240


def _edge_proj_body(*refs):
    nout = (len(refs) - 5) // 3
    attr_ref, w1_ref, b1_ref, w2_ref, b2_ref = refs[:5]
    wh_refs = refs[5:5 + nout]
    bh_refs = refs[5 + nout:5 + 2 * nout]
    out_refs = refs[5 + 2 * nout:]
    h = jnp.maximum(
        jnp.dot(attr_ref[...], w1_ref[...], preferred_element_type=jnp.float32)
        + b1_ref[...][None, :],
        0.0,
    )
    ea = (
        jnp.dot(h, w2_ref[...], preferred_element_type=jnp.float32)
        + b2_ref[...][None, :]
    )
    for wh_ref, bh_ref, out_ref in zip(wh_refs, bh_refs, out_refs):
        for c in range(2):
            out_ref[c, :, :] = (
                jnp.dot(ea, wh_ref[c], preferred_element_type=jnp.float32)
                + bh_ref[c][None, :]
            )


def _edge_proj(edge_attr, em, whs, bhs, block=3200):
    e, de = edge_attr.shape
    h = em[0]["w"].shape[1]
    w3 = whs[0].shape[2]
    nout = len(whs)
    return pl.pallas_call(
        _edge_proj_body,
        grid=(e // block,),
        in_specs=[
            pl.BlockSpec((block, de), lambda i: (i, 0)),
            pl.BlockSpec((de, h), lambda i: (0, 0)),
            pl.BlockSpec((h,), lambda i: (0,)),
            pl.BlockSpec((h, h), lambda i: (0, 0)),
            pl.BlockSpec((h,), lambda i: (0,)),
        ]
        + [pl.BlockSpec((2, h, w3), lambda i: (0, 0, 0))] * nout
        + [pl.BlockSpec((2, w3), lambda i: (0, 0))] * nout,
        out_specs=[pl.BlockSpec((2, block, w3), lambda i: (0, i, 0))] * nout,
        out_shape=[jax.ShapeDtypeStruct((2, e, w3), jnp.float32)] * nout,
    )(edge_attr, em[0]["w"], em[0]["b"], em[1]["w"], em[1]["b"], *whs, *bhs)


@functools.cache
def _make_sc_agg(n, e, w):
    w3 = 3 * w
    B = 80
    TILES = 16
    epw = e // TILES
    nchunks = epw // B
    npt = n // TILES
    ZR = 64
    assert epw * TILES == e and nchunks * B == epw and nchunks % 2 == 0
    assert nchunks >= 6
    assert npt * TILES == n and npt % ZR == 0 and npt % 8 == 0

    mesh = plsc.VectorSubcoreMesh(core_axis_name="c", subcore_axis_name="s")

    @functools.partial(
        pl.kernel,
        out_type=jax.ShapeDtypeStruct((2 * n, w3), jnp.float32),
        mesh=mesh,
        compiler_params=pltpu.CompilerParams(use_tc_tiling_on_sc=False),
        scratch_types=[
            pltpu.VMEM((B,), jnp.int32),
            pltpu.VMEM((B,), jnp.int32),
            pltpu.VMEM((B,), jnp.int32),
            pltpu.VMEM((B,), jnp.int32),
            pltpu.VMEM((B,), jnp.int32),
            pltpu.VMEM((B,), jnp.int32),
            pltpu.VMEM((B, w), jnp.float32),
            pltpu.VMEM((B, w), jnp.float32),
            pltpu.VMEM((B, w3), jnp.float32),
            pltpu.VMEM((B, w3), jnp.float32),
            pltpu.VMEM((B, w3), jnp.float32),
            pltpu.VMEM((B, w3), jnp.float32),
            pltpu.VMEM((ZR, w3), jnp.float32),
            pltpu.VMEM_SHARED((n, w3), jnp.float32),
        ] + [pltpu.SemaphoreType.DMA] * 10,
    )
    def sc_agg(xh, eh, src, dst, out, sv0, sv1, dv0, dv1, iv0, iv1,
               xv0, xv1, ev0, ev1, mv0, mv1, zbuf, agg_sh,
               s_src0, s_src1, s_dst0, s_dst1, s_g0, s_g1, s_e0, s_e1,
               s_s0, s_s1):
        c = lax.axis_index("c")
        s = lax.axis_index("s")
        cn = c * n
        src_v, dst_v, idx_v = (sv0, sv1), (dv0, dv1), (iv0, iv1)
        xs_v, e_v, m_v = (xv0, xv1), (ev0, ev1), (mv0, mv1)
        s_src, s_dst = (s_src0, s_src1), (s_dst0, s_dst1)
        s_g, s_e, s_s = (s_g0, s_g1), (s_e0, s_e1), (s_s0, s_s1)

        def src_sl(k):
            return src.at[pl.ds(s * epw + k * B, B)]

        def dst_sl(k):
            return dst.at[pl.ds(s * epw + k * B, B)]

        def eh_sl(k):
            return eh.at[pl.ds(c * e + s * epw + k * B, B)]

        def zrow(r, carry):
            for g in range(w3 // 16):
                zbuf[r, pl.ds(g * 16, 16)] = jnp.zeros((16,), jnp.float32)
            return carry

        lax.fori_loop(0, ZR, zrow, 0)
        row0 = s * npt
        for j in range(npt // ZR):
            pltpu.sync_copy(zbuf, agg_sh.at[pl.ds(row0 + j * ZR, ZR)])
        plsc.subcore_barrier()

        def start_gather(k, p):
            for g in range(B // 16):
                idx_v[p][pl.ds(g * 16, 16)] = (
                    src_v[p][pl.ds(g * 16, 16)] + cn
                )
            pltpu.async_copy(xh.at[idx_v[p]], xs_v[p], s_g[p])
            pltpu.async_copy(eh_sl(k), e_v[p], s_e[p])

        def compute(p):
            def edge_row(b, carry2):
                for g in range(w // 16):
                    xv = xs_v[p][b, pl.ds(g * 16, 16)]
                    for i in range(3):
                        col = i * w + g * 16
                        m_v[p][b, pl.ds(col, 16)] = jnp.maximum(
                            xv + e_v[p][b, pl.ds(col, 16)], 0.0
                        )
                return carry2

            lax.fori_loop(0, B, edge_row, 0)

        def wait_scatter(p):
            pltpu.make_async_copy(m_v[p], agg_sh.at[pl.ds(0, B)],
                                  s_s[p]).wait()

        def body(k, p, first2=False, pre1=True, pre2=True):
            p1 = 1 - p
            if pre1:
                pltpu.make_async_copy(src_sl(0), src_v[p1], s_src[p1]).wait()
                start_gather(k + 1, p1)
            if not first2:
                wait_scatter(p)
                pltpu.async_copy(dst_sl(k), dst_v[p], s_dst[p])
            pltpu.make_async_copy(xh.at[pl.ds(0, B)], xs_v[p], s_g[p]).wait()
            pltpu.make_async_copy(eh_sl(0), e_v[p], s_e[p]).wait()
            compute(p)
            pltpu.make_async_copy(dst_sl(0), dst_v[p], s_dst[p]).wait()
            pltpu.async_copy(m_v[p], agg_sh.at[dst_v[p]], s_s[p], add=True)
            if pre2:
                pltpu.async_copy(src_sl(k + 2), src_v[p], s_src[p])

        pltpu.async_copy(src_sl(0), src_v[0], s_src[0])
        pltpu.async_copy(src_sl(1), src_v[1], s_src[1])
        pltpu.async_copy(dst_sl(0), dst_v[0], s_dst[0])
        pltpu.async_copy(dst_sl(1), dst_v[1], s_dst[1])
        pltpu.make_async_copy(src_sl(0), src_v[0], s_src[0]).wait()
        start_gather(0, 0)
        body(0, 0, first2=True)
        body(1, 1, first2=True)

        def group(g, carry):
            body(2 * g, 0)
            body(2 * g + 1, 1)
            return carry

        lax.fori_loop(1, nchunks // 2 - 1, group, 0)
        body(nchunks - 2, 0, pre2=False)
        body(nchunks - 1, 1, pre1=False, pre2=False)
        wait_scatter(0)
        wait_scatter(1)
        plsc.subcore_barrier()

        pltpu.sync_copy(agg_sh.at[pl.ds(row0, npt)],
                        out.at[pl.ds(cn + row0, npt)])

    return sc_agg


def _apply_lin(p, x):
    return x @ p["w"] + p["b"]


def _mlp(ps, x):
    return _apply_lin(ps[1], jnp.maximum(_apply_lin(ps[0], x), 0.0))


def _col_group_weights(convs, q, w):
    wh = jnp.concatenate(
        [convs[i]["lin"]["w"][:, q * w:(q + 1) * w] for i in range(3)], axis=1
    )
    bh = jnp.concatenate(
        [convs[i]["lin"]["b"][q * w:(q + 1) * w] for i in range(3)]
    )
    return wh, bh


def _gine_layer(x, src, dst, edge_attr, em, convs, lin_out):
    n, d = x.shape
    e = src.shape[0]
    w = 32
    ngrp = d // w
    ncall = ngrp // 2
    npad = _NPAD
    whs, bhs = [], []
    for r in range(ncall):
        pair = [_col_group_weights(convs, 2 * r + c, w) for c in range(2)]
        whs.append(jnp.stack([p[0] for p in pair]))
        bhs.append(jnp.stack([p[1] for p in pair]))
    ehs = _edge_proj(edge_attr, em, whs, bhs)
    if ncall == 1:
        ehs = [ehs] if not isinstance(ehs, (list, tuple)) else ehs
    xp = jnp.pad(x, ((0, npad - n), (0, 0)))
    xq = xp.reshape(npad, ngrp, w).transpose(1, 0, 2)
    sc = _make_sc_agg(npad, e, w)
    aggs = []
    for r in range(ncall):
        xh = xq[2 * r:2 * r + 2].reshape(2 * npad, w)
        aggs.append(sc(xh, ehs[r].reshape(2 * e, 3 * w), src, dst))
    outs = []
    for i in range(3):
        agg_i = jnp.concatenate(
            [
                aggs[q // 2][(q % 2) * npad:(q % 2) * npad + n,
                             i * w:(i + 1) * w]
                for q in range(ngrp)
            ],
            axis=1,
        )
        outs.append(_mlp(convs[i]["nn"], x + agg_i))
    h = jnp.concatenate(outs, axis=1)
    return jnp.maximum(_apply_lin(lin_out, h), 0.0)


def kernel(x, edge_index, edge_attr, u, params):
    src = edge_index[0]
    dst = edge_index[1]
    x1 = _gine_layer(x, src, dst, edge_attr, params["em1"], params["c1"],
                     params["lin1"])
    x2 = _gine_layer(x1, src, dst, edge_attr, params["em2"], params["c2"],
                     params["lin2"])
    pooled = jnp.mean(x2, axis=0, keepdims=True)
    return _apply_lin(params["fc"], jnp.concatenate([pooled, u], axis=1))
